# baseline (speedup 1.0000x reference)
import functools

import jax
import jax.numpy as jnp
import numpy as np
from jax import lax
from jax.experimental import pallas as pl
from jax.experimental.pallas import tpu as pltpu
from jax.experimental.pallas import tpu_sc as plsc

EMB_SIZE = 1000000
D = 64
B = 16384
K = 20
L = 16

NC = 2
NS = 16
NW = NC * NS
BW = B // NW
CB = 32
NCHUNK = BW // CB
TRG_CHUNK = CB * K
NGATHER = TRG_CHUNK // 128
TROWS = BW * K // 128
SROWS = BW // 128
PROWS = TRG_CHUNK * L // 128

_mesh = plsc.VectorSubcoreMesh(core_axis_name="c", subcore_axis_name="s")


@functools.partial(
    pl.kernel,
    out_type=jax.ShapeDtypeStruct((B * K * L // 128, 128), jnp.float32),
    mesh=_mesh,
    scratch_types=[
        pltpu.VMEM((SROWS, 128), jnp.int32),
        pltpu.VMEM((TROWS, 128), jnp.int32),
        pltpu.VMEM((BW // 4, 128), jnp.int32),
        pltpu.VMEM((CB, 128), jnp.float32),
        pltpu.VMEM((TRG_CHUNK, 128), jnp.float32),
        pltpu.VMEM((PROWS, 128), jnp.float32),
        pltpu.SemaphoreType.DMA,
    ],
)
def _sc_pred(src_idx_hbm, trg_idx_hbm, lsb_hbm,
             u2_hbm, v2_hbm, part_hbm,
             idx_src_v, idx_trg_v, lsb_v,
             src_buf, trg_buf, part_buf, sem):
    c = lax.axis_index("c")
    s = lax.axis_index("s")
    w = s * NC + c
    pltpu.sync_copy(src_idx_hbm.at[w], idx_src_v)
    pltpu.sync_copy(trg_idx_hbm.at[w], idx_trg_v)
    pltpu.sync_copy(lsb_hbm.at[w], lsb_v)

    def chunk_body(ch, carry):
        copies = [pltpu.async_copy(
            u2_hbm.at[idx_src_v.at[ch >> 2].at[pl.ds((ch & 3) * CB, CB)]],
            src_buf, sem)]
        for j in range(NGATHER):
            copies.append(pltpu.async_copy(
                v2_hbm.at[idx_trg_v.at[ch * NGATHER + j]],
                trg_buf.at[pl.ds(j * 128, 128)], sem))
        for cp in copies:
            cp.wait()

        def i_body(i, carry2):
            g = ch * CB + i
            c0 = (g & 3) * 32
            lv0 = lsb_v[g >> 2, pl.ds(c0, L)]
            lv1 = lsb_v[g >> 2, pl.ds(c0 + L, L)]
            soff = lv1[K - L]
            sv = [src_buf[i, pl.ds(soff + q * L, L)] for q in range(D // L)]
            row0 = i * K
            for k in range(K):
                row = row0 + k
                toff = lv0[k] if k < L else lv1[k - L]
                acc = trg_buf[row, pl.ds(toff, L)] * sv[0]
                for q in range(1, D // L):
                    acc = acc + trg_buf[row, pl.ds(toff + q * L, L)] * sv[q]
                part_buf[row >> 3, pl.ds((row & 7) * L, L)] = acc
            return carry2

        lax.fori_loop(0, CB, i_body, 0)

        base = w * (BW * K * L // 128) + ch * PROWS
        pltpu.sync_copy(part_buf, part_hbm.at[pl.ds(base, PROWS)])
        return carry

    lax.fori_loop(0, NCHUNK, chunk_body, 0)


_TC_BB = 2048


def _tc_body(p_ref, m_ref, w_ref, y_ref, o_ref):
    @pl.when(pl.program_id(0) == 0)
    def _():
        o_ref[0, 0] = 0.0

    pred = jnp.dot(p_ref[...], m_ref[...],
                   preferred_element_type=jnp.float32)
    w = w_ref[...]
    y = y_ref[...]
    bce = jnp.maximum(pred, 0.0) - pred * y + jnp.log1p(jnp.exp(-jnp.abs(pred)))
    num = jnp.sum(w * bce, axis=1)
    den = jnp.sum(w, axis=1)
    o_ref[0, 0] += jnp.sum(num / den) * (1.0 / B)


def _tc_loss(partials, wmasks, labels):
    m = jnp.asarray(np.kron(np.eye(K, dtype=np.float32),
                            np.ones((L, 1), np.float32)))
    out = pl.pallas_call(
        _tc_body,
        grid=(B // _TC_BB,),
        in_specs=[
            pl.BlockSpec((_TC_BB, K * L), lambda i: (i, 0)),
            pl.BlockSpec((K * L, K), lambda i: (0, 0)),
            pl.BlockSpec((_TC_BB, K), lambda i: (i, 0)),
            pl.BlockSpec((_TC_BB, K), lambda i: (i, 0)),
        ],
        out_specs=pl.BlockSpec(memory_space=pltpu.SMEM),
        out_shape=jax.ShapeDtypeStruct((1, 1), jnp.float32),
    )(partials, m, wmasks, labels)
    return out[0, 0]


def kernel(src_words, trg_words, wmasks, labels, u_emb, v_emb):
    src_i = src_words.astype(jnp.int32)
    trg_i = trg_words.astype(jnp.int32)
    src_idx = (src_i >> 1).reshape(NW, SROWS, 128)
    trg_idx = (trg_i >> 1).reshape(NW, TROWS, 128)
    lsb = jnp.concatenate(
        [(trg_i & 1) * D, (src_i & 1)[:, None] * D,
         jnp.zeros((B, 32 - K - 1), jnp.int32)], axis=1)
    lsb = lsb.reshape(NW, BW // 4, 128)
    u2 = u_emb.reshape(EMB_SIZE // 2, 2 * D)
    v2 = v_emb.reshape(EMB_SIZE // 2, 2 * D)
    partials = _sc_pred(src_idx, trg_idx, lsb, u2, v2)
    partials = partials.reshape(B, K * L)
    return _tc_loss(partials, wmasks, labels)


# speedup vs baseline: 1.0003x; 1.0003x over previous
import functools

import jax
import jax.numpy as jnp
import numpy as np
from jax import lax
from jax.experimental import pallas as pl
from jax.experimental.pallas import tpu as pltpu
from jax.experimental.pallas import tpu_sc as plsc

EMB_SIZE = 1000000
D = 64
B = 16384
K = 20
L = 16

NC = 2
NS = 16
NW = NC * NS
BW = B // NW
CB = 32
NCHUNK = BW // CB
TRG_CHUNK = CB * K
NGATHER = TRG_CHUNK // 128
TROWS = BW * K // 128
SROWS = BW // 128
PROWS = TRG_CHUNK * L // 128

_mesh = plsc.VectorSubcoreMesh(core_axis_name="c", subcore_axis_name="s")


@functools.partial(
    pl.kernel,
    out_type=jax.ShapeDtypeStruct((B * K * L // 128, 128), jnp.float32),
    mesh=_mesh,
    scratch_types=[
        pltpu.VMEM((SROWS, 128), jnp.int32),
        pltpu.VMEM((TROWS, 128), jnp.int32),
        pltpu.VMEM((BW // 4, 128), jnp.int32),
        pltpu.VMEM((CB, 128), jnp.float32),
        pltpu.VMEM((TRG_CHUNK, 128), jnp.float32),
        pltpu.VMEM((PROWS, 128), jnp.float32),
        pltpu.SemaphoreType.DMA,
    ],
)
def _sc_pred(src_idx_hbm, trg_idx_hbm, lsb_hbm,
             u2_hbm, v2_hbm, part_hbm,
             idx_src_v, idx_trg_v, lsb_v,
             src_buf, trg_buf, part_buf, sem):
    c = lax.axis_index("c")
    s = lax.axis_index("s")
    w = s * NC + c
    pltpu.sync_copy(src_idx_hbm.at[w], idx_src_v)
    pltpu.sync_copy(trg_idx_hbm.at[w], idx_trg_v)
    pltpu.sync_copy(lsb_hbm.at[w], lsb_v)

    def chunk_body(ch, carry):
        copies = []
        for j in range(CB // L):
            iv = idx_src_v[ch >> 2, pl.ds((ch & 3) * CB + j * L, L)]
            copies.append(pltpu.async_copy(
                u2_hbm.at[iv], src_buf.at[pl.ds(j * L, L)], sem))
        for m in range(TRG_CHUNK // L):
            iv = idx_trg_v[ch * NGATHER + (m >> 3), pl.ds((m & 7) * L, L)]
            copies.append(pltpu.async_copy(
                v2_hbm.at[iv], trg_buf.at[pl.ds(m * L, L)], sem))
        for cp in copies:
            cp.wait()

        def i_body(i, carry2):
            g = ch * CB + i
            c0 = (g & 3) * 32
            lv0 = lsb_v[g >> 2, pl.ds(c0, L)]
            lv1 = lsb_v[g >> 2, pl.ds(c0 + L, L)]
            soff = lv1[K - L]
            sv = [src_buf[i, pl.ds(soff + q * L, L)] for q in range(D // L)]
            row0 = i * K
            for k in range(K):
                row = row0 + k
                toff = lv0[k] if k < L else lv1[k - L]
                acc = trg_buf[row, pl.ds(toff, L)] * sv[0]
                for q in range(1, D // L):
                    acc = acc + trg_buf[row, pl.ds(toff + q * L, L)] * sv[q]
                part_buf[row >> 3, pl.ds((row & 7) * L, L)] = acc
            return carry2

        lax.fori_loop(0, CB, i_body, 0)

        base = w * (BW * K * L // 128) + ch * PROWS
        pltpu.sync_copy(part_buf, part_hbm.at[pl.ds(base, PROWS)])
        return carry

    lax.fori_loop(0, NCHUNK, chunk_body, 0)


_TC_BB = 2048


def _tc_body(p_ref, m_ref, w_ref, y_ref, o_ref):
    @pl.when(pl.program_id(0) == 0)
    def _():
        o_ref[0, 0] = 0.0

    pred = jnp.dot(p_ref[...], m_ref[...],
                   preferred_element_type=jnp.float32)
    w = w_ref[...]
    y = y_ref[...]
    bce = jnp.maximum(pred, 0.0) - pred * y + jnp.log1p(jnp.exp(-jnp.abs(pred)))
    num = jnp.sum(w * bce, axis=1)
    den = jnp.sum(w, axis=1)
    o_ref[0, 0] += jnp.sum(num / den) * (1.0 / B)


def _tc_loss(partials, wmasks, labels):
    m = jnp.asarray(np.kron(np.eye(K, dtype=np.float32),
                            np.ones((L, 1), np.float32)))
    out = pl.pallas_call(
        _tc_body,
        grid=(B // _TC_BB,),
        in_specs=[
            pl.BlockSpec((_TC_BB, K * L), lambda i: (i, 0)),
            pl.BlockSpec((K * L, K), lambda i: (0, 0)),
            pl.BlockSpec((_TC_BB, K), lambda i: (i, 0)),
            pl.BlockSpec((_TC_BB, K), lambda i: (i, 0)),
        ],
        out_specs=pl.BlockSpec(memory_space=pltpu.SMEM),
        out_shape=jax.ShapeDtypeStruct((1, 1), jnp.float32),
    )(partials, m, wmasks, labels)
    return out[0, 0]


def kernel(src_words, trg_words, wmasks, labels, u_emb, v_emb):
    src_i = src_words.astype(jnp.int32)
    trg_i = trg_words.astype(jnp.int32)
    src_idx = (src_i >> 1).reshape(NW, SROWS, 128)
    trg_idx = (trg_i >> 1).reshape(NW, TROWS, 128)
    lsb = jnp.concatenate(
        [(trg_i & 1) * D, (src_i & 1)[:, None] * D,
         jnp.zeros((B, 32 - K - 1), jnp.int32)], axis=1)
    lsb = lsb.reshape(NW, BW // 4, 128)
    u2 = u_emb.reshape(EMB_SIZE // 2, 2 * D)
    v2 = v_emb.reshape(EMB_SIZE // 2, 2 * D)
    partials = _sc_pred(src_idx, trg_idx, lsb, u2, v2)
    partials = partials.reshape(B, K * L)
    return _tc_loss(partials, wmasks, labels)


# speedup vs baseline: 1.0010x; 1.0006x over previous
import functools

import jax
import jax.numpy as jnp
import numpy as np
from jax import lax
from jax.experimental import pallas as pl
from jax.experimental.pallas import tpu as pltpu
from jax.experimental.pallas import tpu_sc as plsc

EMB_SIZE = 1000000
D = 64
B = 16384
K = 20
L = 16

NC = 2
NS = 16
NW = NC * NS
BW = B // NW
CB = 32
NCHUNK = BW // CB
TRG_CHUNK = CB * K
NGATHER = TRG_CHUNK // 128
TROWS = BW * K // 128
SROWS = BW // 128
PROWS = TRG_CHUNK * L // 128

_mesh = plsc.VectorSubcoreMesh(core_axis_name="c", subcore_axis_name="s")


@functools.partial(
    pl.kernel,
    out_type=jax.ShapeDtypeStruct((B * K * L // 128, 128), jnp.float32),
    mesh=_mesh,
    compiler_params=pltpu.CompilerParams(use_tc_tiling_on_sc=False),
    scratch_types=[
        pltpu.VMEM((SROWS, 128), jnp.int32),
        pltpu.VMEM((TROWS, 128), jnp.int32),
        pltpu.VMEM((BW // 4, 128), jnp.int32),
        pltpu.VMEM((CB, 128), jnp.float32),
        pltpu.VMEM((TRG_CHUNK, 128), jnp.float32),
        pltpu.VMEM((PROWS, 128), jnp.float32),
        pltpu.SemaphoreType.DMA,
    ],
)
def _sc_pred(src_idx_hbm, trg_idx_hbm, lsb_hbm,
             u2_hbm, v2_hbm, part_hbm,
             idx_src_v, idx_trg_v, lsb_v,
             src_buf, trg_buf, part_buf, sem):
    c = lax.axis_index("c")
    s = lax.axis_index("s")
    w = s * NC + c
    pltpu.sync_copy(src_idx_hbm.at[w], idx_src_v)
    pltpu.sync_copy(trg_idx_hbm.at[w], idx_trg_v)
    pltpu.sync_copy(lsb_hbm.at[w], lsb_v)

    def chunk_body(ch, carry):
        copies = []
        for j in range(CB // L):
            iv = idx_src_v[ch >> 2, pl.ds((ch & 3) * CB + j * L, L)]
            copies.append(pltpu.async_copy(
                u2_hbm.at[iv], src_buf.at[pl.ds(j * L, L)], sem))
        for m in range(TRG_CHUNK // L):
            iv = idx_trg_v[ch * NGATHER + (m >> 3), pl.ds((m & 7) * L, L)]
            copies.append(pltpu.async_copy(
                v2_hbm.at[iv], trg_buf.at[pl.ds(m * L, L)], sem))
        for cp in copies:
            cp.wait()

        def i_body(i, carry2):
            g = ch * CB + i
            c0 = (g & 3) * 32
            lv0 = lsb_v[g >> 2, pl.ds(c0, L)]
            lv1 = lsb_v[g >> 2, pl.ds(c0 + L, L)]
            soff = lv1[K - L]
            sv = [src_buf[i, pl.ds(soff + q * L, L)] for q in range(D // L)]
            row0 = i * K
            for k in range(K):
                row = row0 + k
                toff = lv0[k] if k < L else lv1[k - L]
                acc = trg_buf[row, pl.ds(toff, L)] * sv[0]
                for q in range(1, D // L):
                    acc = acc + trg_buf[row, pl.ds(toff + q * L, L)] * sv[q]
                part_buf[row >> 3, pl.ds((row & 7) * L, L)] = acc
            return carry2

        lax.fori_loop(0, CB, i_body, 0)

        base = w * (BW * K * L // 128) + ch * PROWS
        pltpu.sync_copy(part_buf, part_hbm.at[pl.ds(base, PROWS)])
        return carry

    lax.fori_loop(0, NCHUNK, chunk_body, 0)


_TC_BB = 2048


def _tc_body(p_ref, m_ref, w_ref, y_ref, o_ref):
    @pl.when(pl.program_id(0) == 0)
    def _():
        o_ref[0, 0] = 0.0

    pred = jnp.dot(p_ref[...], m_ref[...],
                   preferred_element_type=jnp.float32)
    w = w_ref[...]
    y = y_ref[...]
    bce = jnp.maximum(pred, 0.0) - pred * y + jnp.log1p(jnp.exp(-jnp.abs(pred)))
    num = jnp.sum(w * bce, axis=1)
    den = jnp.sum(w, axis=1)
    o_ref[0, 0] += jnp.sum(num / den) * (1.0 / B)


def _tc_loss(partials, wmasks, labels):
    m = jnp.asarray(np.kron(np.eye(K, dtype=np.float32),
                            np.ones((L, 1), np.float32)))
    out = pl.pallas_call(
        _tc_body,
        grid=(B // _TC_BB,),
        in_specs=[
            pl.BlockSpec((_TC_BB, K * L), lambda i: (i, 0)),
            pl.BlockSpec((K * L, K), lambda i: (0, 0)),
            pl.BlockSpec((_TC_BB, K), lambda i: (i, 0)),
            pl.BlockSpec((_TC_BB, K), lambda i: (i, 0)),
        ],
        out_specs=pl.BlockSpec(memory_space=pltpu.SMEM),
        out_shape=jax.ShapeDtypeStruct((1, 1), jnp.float32),
    )(partials, m, wmasks, labels)
    return out[0, 0]


def kernel(src_words, trg_words, wmasks, labels, u_emb, v_emb):
    src_i = src_words.astype(jnp.int32)
    trg_i = trg_words.astype(jnp.int32)
    src_idx = (src_i >> 1).reshape(NW, SROWS, 128)
    trg_idx = (trg_i >> 1).reshape(NW, TROWS, 128)
    lsb = jnp.concatenate(
        [(trg_i & 1) * D, (src_i & 1)[:, None] * D,
         jnp.zeros((B, 32 - K - 1), jnp.int32)], axis=1)
    lsb = lsb.reshape(NW, BW // 4, 128)
    u2 = u_emb.reshape(EMB_SIZE // 2, 2 * D)
    v2 = v_emb.reshape(EMB_SIZE // 2, 2 * D)
    partials = _sc_pred(src_idx, trg_idx, lsb, u2, v2)
    partials = partials.reshape(B, K * L)
    return _tc_loss(partials, wmasks, labels)


# speedup vs baseline: 1.0693x; 1.0683x over previous
import functools

import jax
import jax.numpy as jnp
import numpy as np
from jax import lax
from jax.experimental import pallas as pl
from jax.experimental.pallas import tpu as pltpu
from jax.experimental.pallas import tpu_sc as plsc

EMB_SIZE = 1000000
D = 64
B = 16384
K = 20
L = 16

NC = 2
NS = 16
NW = NC * NS
BW = B // NW
CB = 32
NCHUNK = BW // CB
TRG_CHUNK = CB * K
NGATHER = TRG_CHUNK // 128
TROWS = BW * K // 128

_mesh = plsc.VectorSubcoreMesh(core_axis_name="c", subcore_axis_name="s")


@functools.partial(
    pl.kernel,
    out_type=jax.ShapeDtypeStruct((B * K, L), jnp.float32),
    mesh=_mesh,
    compiler_params=pltpu.CompilerParams(use_tc_tiling_on_sc=False),
    scratch_types=[
        pltpu.VMEM((NCHUNK, CB), jnp.int32),
        pltpu.VMEM((TROWS, 128), jnp.int32),
        pltpu.VMEM((2, CB, D), jnp.float32),
        pltpu.VMEM((2, TRG_CHUNK, D), jnp.float32),
        pltpu.VMEM((2, TRG_CHUNK, L), jnp.float32),
        pltpu.SemaphoreType.DMA((2,)),
        pltpu.SemaphoreType.DMA((2,)),
    ],
)
def _sc_pred(src_idx_hbm, trg_idx_hbm, u_hbm, v_hbm, part_hbm,
             idx_src_v, idx_trg_v, src_buf, trg_buf, part_buf, gsem, wsem):
    c = lax.axis_index("c")
    s = lax.axis_index("s")
    w = s * NC + c
    pltpu.sync_copy(src_idx_hbm.at[w], idx_src_v)
    pltpu.sync_copy(trg_idx_hbm.at[w], idx_trg_v)

    def issue_gathers(ch, slot):
        pltpu.async_copy(u_hbm.at[idx_src_v.at[ch]], src_buf.at[slot],
                         gsem.at[slot])
        for j in range(NGATHER):
            pltpu.async_copy(
                v_hbm.at[idx_trg_v.at[ch * NGATHER + j]],
                trg_buf.at[slot].at[pl.ds(j * 128, 128)], gsem.at[slot])

    issue_gathers(0, 0)

    def chunk_body(ch, carry):
        slot = lax.rem(ch, 2)

        @pl.when(ch + 1 < NCHUNK)
        def _():
            issue_gathers(ch + 1, 1 - slot)

        pltpu.make_async_copy(u_hbm.at[pl.ds(0, CB)], src_buf.at[slot],
                              gsem.at[slot]).wait()
        pltpu.make_async_copy(v_hbm.at[pl.ds(0, TRG_CHUNK)],
                              trg_buf.at[slot], gsem.at[slot]).wait()

        base = w * (BW * K) + ch * TRG_CHUNK

        @pl.when(ch >= 2)
        def _():
            pltpu.make_async_copy(part_buf.at[slot],
                                  part_hbm.at[pl.ds(base, TRG_CHUNK)],
                                  wsem.at[slot]).wait()

        sbuf = src_buf.at[slot]
        tbuf = trg_buf.at[slot]
        pbuf = part_buf.at[slot]

        def i_body(i, carry2):
            sv = [sbuf[i, pl.ds(q * L, L)] for q in range(D // L)]
            row0 = i * K
            for k in range(K):
                row = row0 + k
                acc = tbuf[row, pl.ds(0, L)] * sv[0]
                for q in range(1, D // L):
                    acc = acc + tbuf[row, pl.ds(q * L, L)] * sv[q]
                pbuf[row, :] = acc
            return carry2

        lax.fori_loop(0, CB, i_body, 0)

        pltpu.async_copy(pbuf, part_hbm.at[pl.ds(base, TRG_CHUNK)],
                         wsem.at[slot])
        return carry

    lax.fori_loop(0, NCHUNK, chunk_body, 0)

    for slot in range(2):
        pltpu.make_async_copy(part_buf.at[slot],
                              part_hbm.at[pl.ds(0, TRG_CHUNK)],
                              wsem.at[slot]).wait()


_TC_BB = 2048


def _tc_body(p_ref, m_ref, w_ref, y_ref, o_ref):
    @pl.when(pl.program_id(0) == 0)
    def _():
        o_ref[0, 0] = 0.0

    pred = jnp.dot(p_ref[...], m_ref[...],
                   preferred_element_type=jnp.float32)
    w = w_ref[...]
    y = y_ref[...]
    bce = jnp.maximum(pred, 0.0) - pred * y + jnp.log1p(jnp.exp(-jnp.abs(pred)))
    num = jnp.sum(w * bce, axis=1)
    den = jnp.sum(w, axis=1)
    o_ref[0, 0] += jnp.sum(num / den) * (1.0 / B)


def _tc_loss(partials, wmasks, labels):
    m = jnp.asarray(np.kron(np.eye(K, dtype=np.float32),
                            np.ones((L, 1), np.float32)))
    out = pl.pallas_call(
        _tc_body,
        grid=(B // _TC_BB,),
        in_specs=[
            pl.BlockSpec((_TC_BB, K * L), lambda i: (i, 0)),
            pl.BlockSpec((K * L, K), lambda i: (0, 0)),
            pl.BlockSpec((_TC_BB, K), lambda i: (i, 0)),
            pl.BlockSpec((_TC_BB, K), lambda i: (i, 0)),
        ],
        out_specs=pl.BlockSpec(memory_space=pltpu.SMEM),
        out_shape=jax.ShapeDtypeStruct((1, 1), jnp.float32),
    )(partials, m, wmasks, labels)
    return out[0, 0]


def kernel(src_words, trg_words, wmasks, labels, u_emb, v_emb):
    src_idx = src_words.astype(jnp.int32).reshape(NW, NCHUNK, CB)
    trg_idx = trg_words.astype(jnp.int32).reshape(NW, TROWS, 128)
    partials = _sc_pred(src_idx, trg_idx, u_emb, v_emb).reshape(B, K * L)
    return _tc_loss(partials, wmasks, labels)
